# baseline (speedup 1.0000x reference)
import functools

import jax
import jax.numpy as jnp
from jax import lax
from jax.experimental import pallas as pl
from jax.experimental.pallas import tpu as pltpu
from jax.experimental.pallas import tpu_sc as plsc

EMBED = 32
LANES = 16
WAVE = 1
PHASES = 8
SUBWAVES = LANES // WAVE
AHEAD = PHASES - 1


@functools.lru_cache(maxsize=None)
def _make_sc_lookup(batch: int):
    info = plsc.get_sparse_core_info()
    nw = info.num_cores * info.num_subcores
    bw = batch // nw
    assert batch % nw == 0 and bw % LANES == 0
    nchunks = bw // LANES
    assert SUBWAVES % PHASES == 0
    mesh = plsc.VectorSubcoreMesh(core_axis_name="c", subcore_axis_name="s")

    @functools.partial(
        pl.kernel,
        mesh=mesh,
        compiler_params=pltpu.CompilerParams(needs_layout_passes=False),
        out_type=jax.ShapeDtypeStruct((2 * EMBED, batch), jnp.float32),
        scratch_types=[
            pltpu.VMEM((bw,), jnp.int32),
            pltpu.VMEM((bw,), jnp.int32),
            pltpu.VMEM((PHASES, WAVE, EMBED, 128), jnp.float32),
            pltpu.VMEM((PHASES, WAVE, EMBED, 128), jnp.float32),
            pltpu.VMEM((EMBED, bw), jnp.float32),
            pltpu.VMEM((EMBED, bw), jnp.float32),
        ] + [pltpu.SemaphoreType.DMA] * (2 * PHASES + 1),
    )
    def sc_lookup(idx_u_hbm, idx_l_hbm, emb_u_hbm, emb_l_hbm, out_hbm,
                  idx_u_v, idx_l_v, buf_u, buf_l, rows_u, rows_l, *sems):
        sems_u = sems[:PHASES]
        sems_l = sems[PHASES:2 * PHASES]
        sem_w = sems[2 * PHASES]
        wid = lax.axis_index("s") * info.num_cores + lax.axis_index("c")
        base = wid * bw
        pltpu.sync_copy(idx_u_hbm.at[pl.ds(base, bw)], idx_u_v)
        pltpu.sync_copy(idx_l_hbm.at[pl.ds(base, bw)], idx_l_v)

        c_lo = lax.iota(jnp.int32, LANES)
        c_hi = c_lo + LANES

        def fire(iu_vec, il_vec, sw, ph):
            for j in range(WAVE):
                k = sw * WAVE + j
                cu = pl.multiple_of((iu_vec[k] >> 7) * 128, 128)
                cl = pl.multiple_of((il_vec[k] >> 7) * 128, 128)
                pltpu.async_copy(
                    emb_u_hbm.at[:, pl.ds(cu, 128)], buf_u.at[ph, j],
                    sems_u[ph])
                pltpu.async_copy(
                    emb_l_hbm.at[:, pl.ds(cl, 128)], buf_l.at[ph, j],
                    sems_l[ph])

        def drain_extract(iu_vec, il_vec, u0, sw, ph):
            for j in range(WAVE):
                pltpu.make_async_copy(
                    emb_u_hbm.at[:, pl.ds(0, 128)], buf_u.at[ph, j],
                    sems_u[ph]).wait()
                pltpu.make_async_copy(
                    emb_l_hbm.at[:, pl.ds(0, 128)], buf_l.at[ph, j],
                    sems_l[ph]).wait()
            for j in range(WAVE):
                k = sw * WAVE + j
                lu = jnp.broadcast_to(iu_vec[k] & 127, (LANES,))
                ll = jnp.broadcast_to(il_vec[k] & 127, (LANES,))
                us = jnp.broadcast_to(u0 + k, (LANES,))
                v0 = plsc.load_gather(buf_u.at[ph, j], [c_lo, lu])
                v1 = plsc.load_gather(buf_u.at[ph, j], [c_hi, lu])
                plsc.store_scatter(rows_u, [c_lo, us], v0)
                plsc.store_scatter(rows_u, [c_hi, us], v1)
                w0 = plsc.load_gather(buf_l.at[ph, j], [c_lo, ll])
                w1 = plsc.load_gather(buf_l.at[ph, j], [c_hi, ll])
                plsc.store_scatter(rows_l, [c_lo, us], w0)
                plsc.store_scatter(rows_l, [c_hi, us], w1)

        iu0 = idx_u_v[pl.ds(0, LANES)]
        il0 = idx_l_v[pl.ds(0, LANES)]
        for g in range(AHEAD):
            fire(iu0, il0, g, g % PHASES)

        def do_chunk(c, carry):
            u0 = c * LANES
            iu_vec = idx_u_v[pl.ds(u0, LANES)]
            il_vec = idx_l_v[pl.ds(u0, LANES)]
            for sw in range(SUBWAVES):
                n = sw + AHEAD
                ph_fire = n % PHASES
                if n < SUBWAVES:
                    fire(iu_vec, il_vec, n, ph_fire)
                else:
                    @pl.when(c + 1 < nchunks)
                    def _():
                        iun = idx_u_v[pl.ds((c + 1) * LANES, LANES)]
                        iln = idx_l_v[pl.ds((c + 1) * LANES, LANES)]
                        fire(iun, iln, n - SUBWAVES, ph_fire)
                drain_extract(iu_vec, il_vec, u0, sw, sw % PHASES)
            return carry

        lax.fori_loop(0, nchunks, do_chunk, 0)

        pltpu.async_copy(
            rows_u, out_hbm.at[pl.ds(0, EMBED), pl.ds(base, bw)], sem_w
        ).wait()
        pltpu.async_copy(
            rows_l, out_hbm.at[pl.ds(EMBED, EMBED), pl.ds(base, bw)], sem_w
        ).wait()

    return sc_lookup


def kernel(user_fea, emb_user, emb_location):
    batch = user_fea.shape[0]
    idx_u = user_fea[:, 0].astype(jnp.int32)
    idx_l = user_fea[:, 1].astype(jnp.int32)
    out_t = _make_sc_lookup(batch)(idx_u, idx_l, emb_user.T, emb_location.T)
    return out_t.T


# speedup vs baseline: 1.0718x; 1.0718x over previous
import functools

import jax
import jax.numpy as jnp
from jax import lax
from jax.experimental import pallas as pl
from jax.experimental.pallas import tpu as pltpu
from jax.experimental.pallas import tpu_sc as plsc

EMBED = 32
LANES = 16
WAVE = 2
PHASES = 4
SUBWAVES = LANES // WAVE
AHEAD = PHASES - 1


@functools.lru_cache(maxsize=None)
def _make_sc_lookup(batch: int):
    info = plsc.get_sparse_core_info()
    nw = info.num_cores * info.num_subcores
    bw = batch // nw
    assert batch % nw == 0 and bw % LANES == 0
    nchunks = bw // LANES
    assert SUBWAVES % PHASES == 0
    mesh = plsc.VectorSubcoreMesh(core_axis_name="c", subcore_axis_name="s")

    @functools.partial(
        pl.kernel,
        mesh=mesh,
        compiler_params=pltpu.CompilerParams(needs_layout_passes=False),
        out_type=jax.ShapeDtypeStruct((2 * EMBED, batch), jnp.float32),
        scratch_types=[
            pltpu.VMEM((bw,), jnp.int32),
            pltpu.VMEM((bw,), jnp.int32),
            pltpu.VMEM((PHASES, WAVE, EMBED, 128), jnp.float32),
            pltpu.VMEM((PHASES, WAVE, EMBED, 128), jnp.float32),
            pltpu.VMEM((EMBED, bw), jnp.float32),
            pltpu.VMEM((EMBED, bw), jnp.float32),
        ] + [pltpu.SemaphoreType.DMA] * (2 * PHASES + 1),
    )
    def sc_lookup(idx_u_hbm, idx_l_hbm, emb_u_hbm, emb_l_hbm, out_hbm,
                  idx_u_v, idx_l_v, buf_u, buf_l, rows_u, rows_l, *sems):
        sems_u = sems[:PHASES]
        sems_l = sems[PHASES:2 * PHASES]
        sem_w = sems[2 * PHASES]
        wid = lax.axis_index("s") * info.num_cores + lax.axis_index("c")
        base = wid * bw
        pltpu.sync_copy(idx_u_hbm.at[pl.ds(base, bw)], idx_u_v)
        pltpu.sync_copy(idx_l_hbm.at[pl.ds(base, bw)], idx_l_v)

        c_lo = lax.iota(jnp.int32, LANES)
        c_hi = c_lo + LANES

        def fire(iu_vec, il_vec, sw, ph):
            for j in range(WAVE):
                k = sw * WAVE + j
                cu = pl.multiple_of((iu_vec[k] >> 7) * 128, 128)
                cl = pl.multiple_of((il_vec[k] >> 7) * 128, 128)
                for t in range(EMBED // 8):
                    pltpu.async_copy(
                        emb_u_hbm.at[pl.ds(8 * t, 8), pl.ds(cu, 128)],
                        buf_u.at[ph, j, pl.ds(8 * t, 8)], sems_u[ph])
                    pltpu.async_copy(
                        emb_l_hbm.at[pl.ds(8 * t, 8), pl.ds(cl, 128)],
                        buf_l.at[ph, j, pl.ds(8 * t, 8)], sems_l[ph])

        def drain_extract(iu_vec, il_vec, u0, sw, ph):
            for j in range(WAVE):
                pltpu.make_async_copy(
                    emb_u_hbm.at[:, pl.ds(0, 128)], buf_u.at[ph, j],
                    sems_u[ph]).wait()
                pltpu.make_async_copy(
                    emb_l_hbm.at[:, pl.ds(0, 128)], buf_l.at[ph, j],
                    sems_l[ph]).wait()
            for j in range(WAVE):
                k = sw * WAVE + j
                lu = jnp.broadcast_to(iu_vec[k] & 127, (LANES,))
                ll = jnp.broadcast_to(il_vec[k] & 127, (LANES,))
                us = jnp.broadcast_to(u0 + k, (LANES,))
                v0 = plsc.load_gather(buf_u.at[ph, j], [c_lo, lu])
                v1 = plsc.load_gather(buf_u.at[ph, j], [c_hi, lu])
                plsc.store_scatter(rows_u, [c_lo, us], v0)
                plsc.store_scatter(rows_u, [c_hi, us], v1)
                w0 = plsc.load_gather(buf_l.at[ph, j], [c_lo, ll])
                w1 = plsc.load_gather(buf_l.at[ph, j], [c_hi, ll])
                plsc.store_scatter(rows_l, [c_lo, us], w0)
                plsc.store_scatter(rows_l, [c_hi, us], w1)

        iu0 = idx_u_v[pl.ds(0, LANES)]
        il0 = idx_l_v[pl.ds(0, LANES)]
        for g in range(AHEAD):
            fire(iu0, il0, g, g % PHASES)

        def do_chunk(c, carry):
            u0 = c * LANES
            iu_vec = idx_u_v[pl.ds(u0, LANES)]
            il_vec = idx_l_v[pl.ds(u0, LANES)]
            for sw in range(SUBWAVES):
                n = sw + AHEAD
                ph_fire = n % PHASES
                if n < SUBWAVES:
                    fire(iu_vec, il_vec, n, ph_fire)
                else:
                    @pl.when(c + 1 < nchunks)
                    def _():
                        iun = idx_u_v[pl.ds((c + 1) * LANES, LANES)]
                        iln = idx_l_v[pl.ds((c + 1) * LANES, LANES)]
                        fire(iun, iln, n - SUBWAVES, ph_fire)
                drain_extract(iu_vec, il_vec, u0, sw, sw % PHASES)
            return carry

        lax.fori_loop(0, nchunks, do_chunk, 0)

        pltpu.async_copy(
            rows_u, out_hbm.at[pl.ds(0, EMBED), pl.ds(base, bw)], sem_w
        ).wait()
        pltpu.async_copy(
            rows_l, out_hbm.at[pl.ds(EMBED, EMBED), pl.ds(base, bw)], sem_w
        ).wait()

    return sc_lookup


def kernel(user_fea, emb_user, emb_location):
    batch = user_fea.shape[0]
    idx_u = user_fea[:, 0].astype(jnp.int32)
    idx_l = user_fea[:, 1].astype(jnp.int32)
    out_t = _make_sc_lookup(batch)(idx_u, idx_l, emb_user.T, emb_location.T)
    return out_t.T


# speedup vs baseline: 1.0759x; 1.0038x over previous
import functools

import jax
import jax.numpy as jnp
from jax import lax
from jax.experimental import pallas as pl
from jax.experimental.pallas import tpu as pltpu
from jax.experimental.pallas import tpu_sc as plsc

EMBED = 32
LANES = 16
WAVE = 2
PHASES = 4
SUBWAVES = LANES // WAVE
AHEAD = PHASES - 1


@functools.lru_cache(maxsize=None)
def _make_sc_lookup(batch: int):
    info = plsc.get_sparse_core_info()
    nw = info.num_cores * info.num_subcores
    bw = batch // nw
    assert batch % nw == 0 and bw % LANES == 0
    nchunks = bw // LANES
    assert SUBWAVES % PHASES == 0
    mesh = plsc.VectorSubcoreMesh(core_axis_name="c", subcore_axis_name="s")

    @functools.partial(
        pl.kernel,
        mesh=mesh,
        compiler_params=pltpu.CompilerParams(needs_layout_passes=False),
        out_type=jax.ShapeDtypeStruct((2 * EMBED, batch), jnp.float32),
        scratch_types=[
            pltpu.VMEM((bw,), jnp.int32),
            pltpu.VMEM((bw,), jnp.int32),
            pltpu.VMEM((PHASES, WAVE, EMBED, 128), jnp.float32),
            pltpu.VMEM((PHASES, WAVE, EMBED, 128), jnp.float32),
            pltpu.VMEM((EMBED, bw), jnp.float32),
            pltpu.VMEM((EMBED, bw), jnp.float32),
        ] + [pltpu.SemaphoreType.DMA] * (2 * PHASES + 1),
    )
    def sc_lookup(idx_u_hbm, idx_l_hbm, emb_u_hbm, emb_l_hbm, out_hbm,
                  idx_u_v, idx_l_v, buf_u, buf_l, rows_u, rows_l, *sems):
        sems_u = sems[:PHASES]
        sems_l = sems[PHASES:2 * PHASES]
        sem_w = sems[2 * PHASES]
        wid = lax.axis_index("s") * info.num_cores + lax.axis_index("c")
        base = wid * bw
        pltpu.sync_copy(idx_u_hbm.at[pl.ds(base, bw)], idx_u_v)
        pltpu.sync_copy(idx_l_hbm.at[pl.ds(base, bw)], idx_l_v)

        c_lo = lax.iota(jnp.int32, LANES)
        c_hi = c_lo + LANES

        def fire(iu_vec, il_vec, sw, ph):
            for j in range(WAVE):
                k = sw * WAVE + j
                cu = pl.multiple_of((iu_vec[k] >> 7) * 128, 128)
                cl = pl.multiple_of((il_vec[k] >> 7) * 128, 128)
                pltpu.async_copy(
                    emb_u_hbm.at[:, pl.ds(cu, 128)], buf_u.at[ph, j],
                    sems_u[ph])
                pltpu.async_copy(
                    emb_l_hbm.at[:, pl.ds(cl, 128)], buf_l.at[ph, j],
                    sems_l[ph])

        def drain_extract(iu_vec, il_vec, u0, sw, ph):
            for j in range(WAVE):
                pltpu.make_async_copy(
                    emb_u_hbm.at[:, pl.ds(0, 128)], buf_u.at[ph, j],
                    sems_u[ph]).wait()
                pltpu.make_async_copy(
                    emb_l_hbm.at[:, pl.ds(0, 128)], buf_l.at[ph, j],
                    sems_l[ph]).wait()
            for j in range(WAVE):
                k = sw * WAVE + j
                lu = jnp.broadcast_to(iu_vec[k] & 127, (LANES,))
                ll = jnp.broadcast_to(il_vec[k] & 127, (LANES,))
                us = jnp.broadcast_to(u0 + k, (LANES,))
                v0 = plsc.load_gather(buf_u.at[ph, j], [c_lo, lu])
                v1 = plsc.load_gather(buf_u.at[ph, j], [c_hi, lu])
                plsc.store_scatter(rows_u, [c_lo, us], v0)
                plsc.store_scatter(rows_u, [c_hi, us], v1)
                w0 = plsc.load_gather(buf_l.at[ph, j], [c_lo, ll])
                w1 = plsc.load_gather(buf_l.at[ph, j], [c_hi, ll])
                plsc.store_scatter(rows_l, [c_lo, us], w0)
                plsc.store_scatter(rows_l, [c_hi, us], w1)

        iu0 = idx_u_v[pl.ds(0, LANES)]
        il0 = idx_l_v[pl.ds(0, LANES)]
        for g in range(AHEAD):
            fire(iu0, il0, g, g % PHASES)

        def do_chunk(c, carry):
            u0 = c * LANES
            iu_vec = idx_u_v[pl.ds(u0, LANES)]
            il_vec = idx_l_v[pl.ds(u0, LANES)]
            for sw in range(SUBWAVES):
                n = sw + AHEAD
                ph_fire = n % PHASES
                if n < SUBWAVES:
                    fire(iu_vec, il_vec, n, ph_fire)
                else:
                    @pl.when(c + 1 < nchunks)
                    def _():
                        iun = idx_u_v[pl.ds((c + 1) * LANES, LANES)]
                        iln = idx_l_v[pl.ds((c + 1) * LANES, LANES)]
                        fire(iun, iln, n - SUBWAVES, ph_fire)
                drain_extract(iu_vec, il_vec, u0, sw, sw % PHASES)
            return carry

        lax.fori_loop(0, nchunks, do_chunk, 0)

        pltpu.async_copy(
            rows_u, out_hbm.at[pl.ds(0, EMBED), pl.ds(base, bw)], sem_w
        ).wait()
        pltpu.async_copy(
            rows_l, out_hbm.at[pl.ds(EMBED, EMBED), pl.ds(base, bw)], sem_w
        ).wait()

    return sc_lookup


def kernel(user_fea, emb_user, emb_location):
    batch = user_fea.shape[0]
    idx_u = user_fea[:, 0].astype(jnp.int32)
    idx_l = user_fea[:, 1].astype(jnp.int32)
    out_t = _make_sc_lookup(batch)(idx_u, idx_l, emb_user.T, emb_location.T)
    return out_t.T
